# Initial kernel scaffold; baseline (speedup 1.0000x reference)
import jax
import jax.numpy as jnp
from jax.experimental import pallas as pl


def kernel(phoneme_ids, table):
    raise NotImplementedError("write your pallas kernel here")


# speedup vs baseline: 1.1479x; 1.1479x over previous
import functools

import jax
import jax.numpy as jnp
from jax import lax
from jax.experimental import pallas as pl
from jax.experimental.pallas import tpu as pltpu
from jax.experimental.pallas import tpu_sc as plsc

VOCAB = 76
DIM = 768
B, L = 64, 1024

NC, NS = 2, 16
NW = NC * NS
ROWS = B * L
ROWS_PER_W = ROWS // NW
CHUNK = 128
NCHUNK = ROWS_PER_W // CHUNK


def _sc_gather(ids_hbm, table_hbm, out_hbm, idx_v, rows_v, sem):
    wid = lax.axis_index("s") * NC + lax.axis_index("c")
    base = wid * ROWS_PER_W
    pltpu.sync_copy(ids_hbm.at[wid], idx_v)

    def chunk_body(c, carry):
        pltpu.async_copy(table_hbm.at[idx_v.at[c]], rows_v, sem).wait()
        pltpu.sync_copy(rows_v, out_hbm.at[pl.ds(base + c * CHUNK, CHUNK)])
        return carry

    lax.fori_loop(0, NCHUNK, chunk_body, 0)


@functools.partial(jax.jit, static_argnames=())
def kernel(phoneme_ids, table):
    ids = phoneme_ids.astype(jnp.int32).reshape(NW, NCHUNK, CHUNK)
    mesh = plsc.VectorSubcoreMesh(
        core_axis_name="c", subcore_axis_name="s", num_cores=NC, num_subcores=NS
    )
    out = pl.kernel(
        _sc_gather,
        out_type=jax.ShapeDtypeStruct((ROWS, DIM), jnp.float32),
        mesh=mesh,
        scratch_types=[
            pltpu.VMEM((NCHUNK, CHUNK), jnp.int32),
            pltpu.VMEM((CHUNK, DIM), jnp.float32),
            pltpu.SemaphoreType.DMA,
        ],
    )(ids, table)
    embeds = out.reshape(B, L, DIM)
    mask = jnp.ones((B, L), dtype=jnp.float32)
    return (embeds, mask)


# speedup vs baseline: 1.1561x; 1.0071x over previous
import functools

import jax
import jax.numpy as jnp
from jax import lax
from jax.experimental import pallas as pl
from jax.experimental.pallas import tpu as pltpu
from jax.experimental.pallas import tpu_sc as plsc

VOCAB = 76
DIM = 768
B, L = 64, 1024

NC, NS = 2, 16
NW = NC * NS
ROWS = B * L
ROWS_PER_W = ROWS // NW
CHUNK = 64
NCHUNK = ROWS_PER_W // CHUNK
NPAIR = NCHUNK // 2


def _sc_gather(ids_hbm, table_hbm, out_hbm, idx_v, buf0, buf1, si0, si1, so0, so1):
    wid = lax.axis_index("s") * NC + lax.axis_index("c")
    base = wid * ROWS_PER_W
    pltpu.sync_copy(ids_hbm.at[wid], idx_v)

    def gather(c, buf, sem):
        return pltpu.make_async_copy(table_hbm.at[idx_v.at[c]], buf, sem)

    def writeout(c, buf, sem):
        return pltpu.make_async_copy(buf, out_hbm.at[pl.ds(base + c * CHUNK, CHUNK)], sem)

    gather(0, buf0, si0).start()

    def pair_body(i, carry):
        c0 = 2 * i
        c1 = c0 + 1
        gather(c0, buf0, si0).wait()
        writeout(c0, buf0, so0).start()

        @pl.when(i > 0)
        def _():
            writeout(c1 - 2, buf1, so1).wait()

        gather(c1, buf1, si1).start()
        gather(c1, buf1, si1).wait()
        writeout(c1, buf1, so1).start()
        writeout(c0, buf0, so0).wait()

        @pl.when(i < NPAIR - 1)
        def _():
            gather(c0 + 2, buf0, si0).start()

        return carry

    lax.fori_loop(0, NPAIR, pair_body, 0)
    writeout(NCHUNK - 1, buf1, so1).wait()


@functools.partial(jax.jit, static_argnames=())
def kernel(phoneme_ids, table):
    ids = phoneme_ids.astype(jnp.int32).reshape(NW, NCHUNK, CHUNK)
    mesh = plsc.VectorSubcoreMesh(
        core_axis_name="c", subcore_axis_name="s", num_cores=NC, num_subcores=NS
    )
    out = pl.kernel(
        _sc_gather,
        out_type=jax.ShapeDtypeStruct((ROWS, DIM), jnp.float32),
        mesh=mesh,
        scratch_types=[
            pltpu.VMEM((NCHUNK, CHUNK), jnp.int32),
            pltpu.VMEM((CHUNK, DIM), jnp.float32),
            pltpu.VMEM((CHUNK, DIM), jnp.float32),
            pltpu.SemaphoreType.DMA,
            pltpu.SemaphoreType.DMA,
            pltpu.SemaphoreType.DMA,
            pltpu.SemaphoreType.DMA,
        ],
    )(ids, table)
    embeds = out.reshape(B, L, DIM)
    mask = jnp.ones((B, L), dtype=jnp.float32)
    return (embeds, mask)


# speedup vs baseline: 2.2616x; 1.9562x over previous
import functools

import jax
import jax.numpy as jnp
from jax import lax
from jax.experimental import pallas as pl
from jax.experimental.pallas import tpu as pltpu
from jax.experimental.pallas import tpu_sc as plsc

VOCAB = 76
DIM = 768
B, L = 64, 1024

NC, NS = 2, 16
NW = NC * NS
ROWS = B * L
ROWS_PER_W = ROWS // NW
CHUNK = 64
NCHUNK = ROWS_PER_W // CHUNK
NPAIR = NCHUNK // 2


def _sc_gather(ids_hbm, table_hbm, out_hbm, idx_v, buf0, buf1, si0, si1, so0, so1):
    wid = lax.axis_index("s") * NC + lax.axis_index("c")
    base = wid * ROWS_PER_W
    pltpu.sync_copy(ids_hbm.at[wid], idx_v)

    def gather(c, buf, sem):
        return pltpu.make_async_copy(table_hbm.at[idx_v.at[c]], buf, sem)

    def writeout(c, buf, sem):
        return pltpu.make_async_copy(buf, out_hbm.at[pl.ds(base + c * CHUNK, CHUNK)], sem)

    gather(0, buf0, si0).start()

    def pair_body(i, carry):
        c0 = 2 * i
        c1 = c0 + 1
        gather(c0, buf0, si0).wait()
        writeout(c0, buf0, so0).start()

        @pl.when(i > 0)
        def _():
            writeout(c1 - 2, buf1, so1).wait()

        gather(c1, buf1, si1).start()
        gather(c1, buf1, si1).wait()
        writeout(c1, buf1, so1).start()
        writeout(c0, buf0, so0).wait()

        @pl.when(i < NPAIR - 1)
        def _():
            gather(c0 + 2, buf0, si0).start()

        return carry

    lax.fori_loop(0, NPAIR, pair_body, 0)
    writeout(NCHUNK - 1, buf1, so1).wait()


@functools.partial(jax.jit, static_argnames=())
def kernel(phoneme_ids, table):
    ids = phoneme_ids.astype(jnp.int32).reshape(NW, NCHUNK, CHUNK)
    ids = ids + (jnp.arange(NW, dtype=jnp.int32) * VOCAB)[:, None, None]
    table_rep = jnp.broadcast_to(table, (NW,) + table.shape).reshape(NW * VOCAB, DIM)
    mesh = plsc.VectorSubcoreMesh(
        core_axis_name="c", subcore_axis_name="s", num_cores=NC, num_subcores=NS
    )
    out = pl.kernel(
        _sc_gather,
        out_type=jax.ShapeDtypeStruct((ROWS, DIM), jnp.float32),
        mesh=mesh,
        scratch_types=[
            pltpu.VMEM((NCHUNK, CHUNK), jnp.int32),
            pltpu.VMEM((CHUNK, DIM), jnp.float32),
            pltpu.VMEM((CHUNK, DIM), jnp.float32),
            pltpu.SemaphoreType.DMA,
            pltpu.SemaphoreType.DMA,
            pltpu.SemaphoreType.DMA,
            pltpu.SemaphoreType.DMA,
        ],
    )(ids, table_rep)
    embeds = out.reshape(B, L, DIM)
    mask = jnp.ones((B, L), dtype=jnp.float32)
    return (embeds, mask)


# speedup vs baseline: 2.3180x; 1.0249x over previous
import functools

import jax
import jax.numpy as jnp
from jax import lax
from jax.experimental import pallas as pl
from jax.experimental.pallas import tpu as pltpu
from jax.experimental.pallas import tpu_sc as plsc

VOCAB = 76
DIM = 768
B, L = 64, 1024

NC, NS = 2, 16
NW = NC * NS
ROWS = B * L
ROWS_PER_W = ROWS // NW
CHUNK = 32
NCHUNK = ROWS_PER_W // CHUNK
NBUF = 4
NGROUP = NCHUNK // NBUF


def _sc_gather(ids_hbm, table_hbm, out_hbm, idx_v, bufs, sg, so):
    wid = lax.axis_index("s") * NC + lax.axis_index("c")
    base = wid * ROWS_PER_W
    pltpu.sync_copy(ids_hbm.at[wid], idx_v)

    def gather(c, b):
        return pltpu.make_async_copy(table_hbm.at[idx_v.at[c]], bufs[b], sg[b])

    def writeout(c, b):
        return pltpu.make_async_copy(
            bufs[b], out_hbm.at[pl.ds(base + c * CHUNK, CHUNK)], so[b]
        )

    for b in range(NBUF - 1):
        gather(b, b).start()

    def group_body(j, carry):
        for b in range(NBUF):
            c = j * NBUF + b
            gather(c, b).wait()
            writeout(c, b).start()
            bp = (b - 1) % NBUF

            @pl.when(c >= 1)
            def _():
                writeout(c - 1, bp).wait()

            @pl.when(c + NBUF - 1 < NCHUNK)
            def _():
                gather(c + NBUF - 1, bp).start()

        return carry

    lax.fori_loop(0, NGROUP, group_body, 0)
    writeout(NCHUNK - 1, NBUF - 1).wait()


@functools.partial(jax.jit, static_argnames=())
def kernel(phoneme_ids, table):
    ids = phoneme_ids.astype(jnp.int32).reshape(NW, NCHUNK, CHUNK)
    ids = ids + (jnp.arange(NW, dtype=jnp.int32) * VOCAB)[:, None, None]
    table_rep = jnp.broadcast_to(table, (NW,) + table.shape).reshape(NW * VOCAB, DIM)
    mesh = plsc.VectorSubcoreMesh(
        core_axis_name="c", subcore_axis_name="s", num_cores=NC, num_subcores=NS
    )
    out = pl.kernel(
        _sc_gather,
        out_type=jax.ShapeDtypeStruct((ROWS, DIM), jnp.float32),
        mesh=mesh,
        scratch_types=[
            pltpu.VMEM((NCHUNK, CHUNK), jnp.int32),
            [pltpu.VMEM((CHUNK, DIM), jnp.float32) for _ in range(NBUF)],
            [pltpu.SemaphoreType.DMA for _ in range(NBUF)],
            [pltpu.SemaphoreType.DMA for _ in range(NBUF)],
        ],
    )(ids, table_rep)
    embeds = out.reshape(B, L, DIM)
    mask = jnp.ones((B, L), dtype=jnp.float32)
    return (embeds, mask)
